# Initial kernel scaffold; baseline (speedup 1.0000x reference)
import jax
import jax.numpy as jnp
from jax.experimental import pallas as pl


def kernel(high_emb, low_emb, hr_mesh_0, hr_mesh_1, hr_g2m_feat, hr_m2m_feat_0, hr_m2m_feat_1, hr_mesh_up_feat_0, lr_mesh_0, lr_g2m_feat, lr_m2m_feat_0, params, hr_g2m_edge_index, hr_m2m_edge_index_0, hr_m2m_edge_index_1, hr_mesh_up_edge_index_0, lr_g2m_edge_index, lr_m2m_edge_index_0):
    raise NotImplementedError("write your pallas kernel here")


# speedup vs baseline: 3.3543x; 3.3543x over previous
import functools

import jax
import jax.numpy as jnp
from jax import lax
from jax.experimental import pallas as pl
from jax.experimental.pallas import tpu as pltpu
from jax.experimental.pallas import tpu_sc as plsc

F32 = jnp.float32
H = 128
NC = 2
NS = 16
NW = NC * NS


def _silu(x):
    return x * (1.0 / (1.0 + jnp.exp(-x)))


def _tc_matmul(x, w):
    n = x.shape[0]
    bn = min(n, 2048)

    def body(x_ref, w_ref, o_ref):
        o_ref[...] = jnp.dot(x_ref[...], w_ref[...], preferred_element_type=F32)

    return pl.pallas_call(
        body,
        grid=(n // bn,),
        in_specs=[
            pl.BlockSpec((bn, H), lambda i: (i, 0)),
            pl.BlockSpec((H, H), lambda i: (0, 0)),
        ],
        out_specs=pl.BlockSpec((bn, H), lambda i: (i, 0)),
        out_shape=jax.ShapeDtypeStruct((n, H), F32),
    )(x, w)


def _tc_edge_mlp(edge, m_prev, gs, gd, w1e, b1, w2, b2):
    e = edge.shape[0]
    be = min(e, 2048)
    has_prev = m_prev is not None

    def body(*refs):
        if has_prev:
            e_ref, mp_ref, gs_ref, gd_ref, w1_ref, b1_ref, w2_ref, b2_ref, o_ref = refs
            x = e_ref[...] + mp_ref[...]
        else:
            e_ref, gs_ref, gd_ref, w1_ref, b1_ref, w2_ref, b2_ref, o_ref = refs
            x = e_ref[...]
        pre = (
            jnp.dot(x, w1_ref[...], preferred_element_type=F32)
            + gs_ref[...]
            + gd_ref[...]
            + b1_ref[...]
        )
        h = _silu(pre)
        o_ref[...] = jnp.dot(h, w2_ref[...], preferred_element_type=F32) + b2_ref[...]

    row_spec = pl.BlockSpec((be, H), lambda i: (i, 0))
    w_spec = pl.BlockSpec((H, H), lambda i: (0, 0))
    b_spec = pl.BlockSpec((1, H), lambda i: (0, 0))
    in_specs = [row_spec] + ([row_spec] if has_prev else []) + [
        row_spec, row_spec, w_spec, b_spec, w_spec, b_spec]
    args = [edge] + ([m_prev] if has_prev else []) + [
        gs, gd, w1e, b1.reshape(1, H), w2, b2.reshape(1, H)]
    return pl.pallas_call(
        body,
        grid=(e // be,),
        in_specs=in_specs,
        out_specs=row_spec,
        out_shape=jax.ShapeDtypeStruct((e, H), F32),
    )(*args)


def _tc_node_mlp(rec, a0, a1, extra, v1r, v1a, c1, v2, c2):
    n = rec.shape[0]
    bn = min(n, 2048)
    has_extra = extra is not None

    def body(*refs):
        if has_extra:
            r_ref, a0_ref, a1_ref, x_ref, v1r_ref, v1a_ref, c1_ref, v2_ref, c2_ref, o_ref = refs
        else:
            r_ref, a0_ref, a1_ref, v1r_ref, v1a_ref, c1_ref, v2_ref, c2_ref, o_ref = refs
        rec_v = r_ref[...]
        aggr = a0_ref[...] + a1_ref[...]
        pre = (
            jnp.dot(rec_v, v1r_ref[...], preferred_element_type=F32)
            + jnp.dot(aggr, v1a_ref[...], preferred_element_type=F32)
            + c1_ref[...]
        )
        out = rec_v + jnp.dot(_silu(pre), v2_ref[...], preferred_element_type=F32) + c2_ref[...]
        if has_extra:
            out = out + x_ref[...]
        o_ref[...] = out

    row_spec = pl.BlockSpec((bn, H), lambda i: (i, 0))
    w_spec = pl.BlockSpec((H, H), lambda i: (0, 0))
    b_spec = pl.BlockSpec((1, H), lambda i: (0, 0))
    in_specs = [row_spec, row_spec, row_spec] + ([row_spec] if has_extra else []) + [
        w_spec, w_spec, b_spec, w_spec, b_spec]
    args = [rec, a0, a1] + ([extra] if has_extra else []) + [
        v1r, v1a, c1.reshape(1, H), v2, c2.reshape(1, H)]
    return pl.pallas_call(
        body,
        grid=(n // bn,),
        in_specs=in_specs,
        out_specs=row_spec,
        out_shape=jax.ShapeDtypeStruct((n, H), F32),
    )(*args)


def _tc_final(rep, lpm):
    (w1, b1), (w2, b2) = lpm
    n = rep.shape[0]
    lat_dim = w2.shape[1]
    half = lat_dim // 2

    def body(r_ref, w1_ref, b1_ref, w2_ref, b2_ref, mu_ref, std_ref):
        h = _silu(jnp.dot(r_ref[...], w1_ref[...], preferred_element_type=F32) + b1_ref[...])
        lat = jnp.dot(h, w2_ref[...], preferred_element_type=F32) + b2_ref[...]
        mu_ref[...] = lat[:, :half]
        raw = lat[:, half:]
        sp = jnp.maximum(raw, 0.0) + jnp.log(1.0 + jnp.exp(-jnp.abs(raw)))
        std_ref[...] = 1e-4 + sp

    return pl.pallas_call(
        body,
        in_specs=[
            pl.BlockSpec((n, H), lambda: (0, 0)),
            pl.BlockSpec((H, H), lambda: (0, 0)),
            pl.BlockSpec((1, H), lambda: (0, 0)),
            pl.BlockSpec((H, lat_dim), lambda: (0, 0)),
            pl.BlockSpec((1, lat_dim), lambda: (0, 0)),
        ],
        out_specs=[
            pl.BlockSpec((n, half), lambda: (0, 0)),
            pl.BlockSpec((n, half), lambda: (0, 0)),
        ],
        out_shape=[
            jax.ShapeDtypeStruct((n, half), F32),
            jax.ShapeDtypeStruct((n, half), F32),
        ],
    )(rep, w1, b1.reshape(1, H), w2, b2.reshape(1, lat_dim))


def _sc_mesh():
    return plsc.VectorSubcoreMesh(
        core_axis_name="c", subcore_axis_name="s", num_cores=NC, num_subcores=NS)


def _sc_gather2(sp, dp, src2d, dst2d):
    e = src2d.shape[0] * 128
    nsub = e // NW // 128

    @functools.partial(
        pl.kernel,
        out_type=(
            jax.ShapeDtypeStruct((e, H), F32),
            jax.ShapeDtypeStruct((e, H), F32),
        ),
        mesh=_sc_mesh(),
        scratch_types=[
            pltpu.VMEM((nsub, 128), jnp.int32),
            pltpu.VMEM((nsub, 128), jnp.int32),
            pltpu.VMEM((128, H), F32),
            pltpu.VMEM((128, H), F32),
            pltpu.SemaphoreType.DMA,
            pltpu.SemaphoreType.DMA,
        ],
    )
    def k(sp_hbm, dp_hbm, src_hbm, dst_hbm, gs_hbm, gd_hbm, si, di, rs, rd, sem1, sem2):
        cid = lax.axis_index("c")
        sid = lax.axis_index("s")
        wid = sid * NC + cid
        row0 = wid * nsub
        pltpu.sync_copy(src_hbm.at[pl.ds(row0, nsub)], si)
        pltpu.sync_copy(dst_hbm.at[pl.ds(row0, nsub)], di)

        def body(j, carry):
            c1 = pltpu.async_copy(sp_hbm.at[si.at[j]], rs, sem1)
            c2 = pltpu.async_copy(dp_hbm.at[di.at[j]], rd, sem2)
            c1.wait()
            c2.wait()
            pltpu.sync_copy(rs, gs_hbm.at[pl.ds((row0 + j) * 128, 128)])
            pltpu.sync_copy(rd, gd_hbm.at[pl.ds((row0 + j) * 128, 128)])
            return carry

        lax.fori_loop(0, nsub, body, 0)

    return k(sp, dp, src2d, dst2d)


def _sc_scatter_add(m, dst2d, n_rec):
    e = dst2d.shape[0] * 128
    nsub = e // NW // 128
    nsub_pc = e // NC // 128
    zrows = n_rec // NS

    @functools.partial(
        pl.kernel,
        out_type=jax.ShapeDtypeStruct((NC * n_rec, H), F32),
        mesh=_sc_mesh(),
        scratch_types=[
            pltpu.VMEM((nsub, 128), jnp.int32),
            pltpu.VMEM((128, H), F32),
            pltpu.VMEM((zrows, H), F32),
            pltpu.VMEM_SHARED((n_rec, H), F32),
        ],
    )
    def k(m_hbm, dst_hbm, out_hbm, di, rows, zbuf, shared):
        cid = lax.axis_index("c")
        sid = lax.axis_index("s")

        def zero_body(i, carry):
            zbuf[i // (H // 16), pl.ds((i % (H // 16)) * 16, 16)] = jnp.zeros((16,), F32)
            return carry

        lax.fori_loop(0, zrows * (H // 16), zero_body, 0)
        pltpu.sync_copy(zbuf, shared.at[pl.ds(sid * zrows, zrows)])
        plsc.subcore_barrier()

        row0 = cid * nsub_pc + sid * nsub
        pltpu.sync_copy(dst_hbm.at[pl.ds(row0, nsub)], di)

        def body(j, carry):
            pltpu.sync_copy(m_hbm.at[pl.ds((row0 + j) * 128, 128)], rows)
            pltpu.sync_copy(rows, shared.at[di.at[j]], add=True)
            return carry

        lax.fori_loop(0, nsub, body, 0)
        plsc.subcore_barrier()

        pltpu.sync_copy(shared.at[pl.ds(sid * zrows, zrows)], zbuf)
        pltpu.sync_copy(zbuf, out_hbm.at[pl.ds(cid * n_rec + sid * zrows, zrows)])

    return k(m, dst2d)


def _gnn_layer(p, send, rec, edge, m_prev, src2d, dst2d, extra=None, need_m=False):
    (w1, b1), (w2, b2) = p["edge"]
    w1e, w1s, w1d = w1[:H], w1[H:2 * H], w1[2 * H:]
    (v1, c1), (v2, c2) = p["node"]
    v1r, v1a = v1[:H], v1[H:]

    sp = _tc_matmul(send, w1s)
    dpp = _tc_matmul(rec, w1d)
    gs, gd = _sc_gather2(sp, dpp, src2d, dst2d)
    m = _tc_edge_mlp(edge, m_prev, gs, gd, w1e, b1, w2, b2)
    n_rec = rec.shape[0]
    ag = _sc_scatter_add(m, dst2d, n_rec)
    rec_new = _tc_node_mlp(rec, ag[:n_rec], ag[n_rec:], extra, v1r, v1a, c1, v2, c2)
    return rec_new, (m if need_m else None)


def _ei2d(ei):
    e = ei.shape[1]
    s = ei[0].astype(jnp.int32).reshape(e // 128, 128)
    d = ei[1].astype(jnp.int32).reshape(e // 128, 128)
    return s, d


def kernel(high_emb, low_emb, hr_mesh_0, hr_mesh_1, hr_g2m_feat, hr_m2m_feat_0,
           hr_m2m_feat_1, hr_mesh_up_feat_0, lr_mesh_0, lr_g2m_feat,
           lr_m2m_feat_0, params, hr_g2m_edge_index, hr_m2m_edge_index_0,
           hr_m2m_edge_index_1, hr_mesh_up_edge_index_0, lr_g2m_edge_index,
           lr_m2m_edge_index_0):
    he = high_emb[0]
    le = low_emb[0]
    hm0 = hr_mesh_0[0]
    hm1 = hr_mesh_1[0]
    g2m_feat = hr_g2m_feat[0]
    m2m0 = hr_m2m_feat_0[0]
    m2m1 = hr_m2m_feat_1[0]
    upf = hr_mesh_up_feat_0[0]
    lm0 = lr_mesh_0[0]
    lrg2m = lr_g2m_feat[0]
    lrm2m = lr_m2m_feat_0[0]

    s_g2m, d_g2m = _ei2d(hr_g2m_edge_index)
    s_m2m0, d_m2m0 = _ei2d(hr_m2m_edge_index_0)
    s_m2m1, d_m2m1 = _ei2d(hr_m2m_edge_index_1)
    s_up, d_up = _ei2d(hr_mesh_up_edge_index_0)
    s_lg2m, d_lg2m = _ei2d(lr_g2m_edge_index)
    s_lm2m, d_lm2m = _ei2d(lr_m2m_edge_index_0)

    P = params

    hr_rep, _ = _gnn_layer(P["g2m"], he, hm0, g2m_feat, None, s_g2m, d_g2m)
    n1, m1 = _gnn_layer(P["intra0"][0], hr_rep, hr_rep, m2m0, None,
                        s_m2m0, d_m2m0, need_m=True)
    rep, _ = _gnn_layer(P["intra0"][1], n1, n1, m2m0, m1, s_m2m0, d_m2m0,
                        extra=le)

    lr_up, _ = _gnn_layer(P["cond_g2m"], le, lm0, lrg2m, None, s_lg2m, d_lg2m)
    rep2, _ = _gnn_layer(P["up0"], rep, hm1, upf, None, s_up, d_up, extra=lr_up)

    n2, m2 = _gnn_layer(P["cond_intra0"][0], lr_up, lr_up, lrm2m, None,
                        s_lm2m, d_lm2m, need_m=True)
    lr_in, _ = _gnn_layer(P["cond_intra0"][1], n2, n2, lrm2m, m2,
                          s_lm2m, d_lm2m)

    n3, m3 = _gnn_layer(P["intra1"][0], rep2, rep2, m2m1, None,
                        s_m2m1, d_m2m1, need_m=True)
    rep3, _ = _gnn_layer(P["intra1"][1], n3, n3, m2m1, m3, s_m2m1, d_m2m1,
                         extra=lr_in)

    mu, std = _tc_final(rep3, P["lpm"])
    return (mu[None], std[None], low_emb, lr_in[None], lr_up[None])
